# Initial kernel scaffold; baseline (speedup 1.0000x reference)
import jax
import jax.numpy as jnp
from jax.experimental import pallas as pl


def kernel(hidden_states, experts_indices, start_indices, end_indices, gate_w, up_w, down_w):
    raise NotImplementedError("write your pallas kernel here")


# speedup vs baseline: 2.8312x; 2.8312x over previous
import functools

import jax
import jax.numpy as jnp
from jax import lax
from jax.experimental import pallas as pl
from jax.experimental.pallas import tpu as pltpu
from jax.experimental.pallas import tpu_sc as plsc

E = 8
H = 1024
F = 3584
N = 4096
T = 256
NT = N // T
MAXP = NT + E - 1
FB = 512
NFB = F // FB
NW = 32
CHUNK = N // NW
SUB = 32
NSUB = CHUNK // SUB
L = 16


def _permute_body(x_hbm, idx_hbm, base_hbm, perm_hbm, pos_hbm,
                  idx_v, base_v, posn_v, row_v, sem):
    cid = lax.axis_index("c")
    sid = lax.axis_index("s")
    wid = sid * 2 + cid
    g0 = wid * CHUNK
    pltpu.sync_copy(idx_hbm.at[pl.ds(g0, CHUNK)], idx_v)
    pltpu.sync_copy(base_hbm.at[wid], base_v)
    lane = lax.broadcasted_iota(jnp.int32, (L,), 0)
    for j in range(CHUNK // L):
        x = idx_v[pl.ds(j * L, L)]
        bx = plsc.load_gather(base_v, [x])
        rank = jnp.zeros((L,), jnp.int32)
        cnt = jnp.zeros((L,), jnp.int32)
        for e in range(E):
            m = x == e
            mi = m.astype(jnp.int32)
            cs = plsc.cumsum(mi)
            rank = rank + jnp.where(m, cs - 1, 0)
            tot = jnp.sum(mi)
            cnt = cnt + jnp.where(lane == e, tot, 0)
        pos = bx + rank
        posn_v[j // 2, pl.ds((j % 2) * L, L)] = pos
        base_v[...] = base_v[...] + cnt
    for c in range(NSUB):
        pltpu.sync_copy(x_hbm.at[pl.ds(g0 + c * SUB, SUB)], row_v)
        pltpu.async_copy(row_v, perm_hbm.at[posn_v.at[c]], sem).wait()
    pltpu.sync_copy(posn_v, pos_hbm.at[wid])


@functools.cache
def _permute():
    return pl.kernel(
        _permute_body,
        out_type=[jax.ShapeDtypeStruct((N, H), jnp.float32),
                  jax.ShapeDtypeStruct((NW, NSUB, SUB), jnp.int32)],
        mesh=plsc.VectorSubcoreMesh(core_axis_name="c", subcore_axis_name="s"),
        scratch_types=[pltpu.VMEM((CHUNK,), jnp.int32),
                       pltpu.VMEM((L,), jnp.int32),
                       pltpu.VMEM((NSUB, SUB), jnp.int32),
                       pltpu.VMEM((SUB, H), jnp.float32),
                       pltpu.SemaphoreType.DMA],
        compiler_params=pltpu.CompilerParams(needs_layout_passes=False),
    )


def _gather_body(moe_hbm, pos2_hbm, out_hbm, pidx_v, row_v, sem):
    cid = lax.axis_index("c")
    sid = lax.axis_index("s")
    wid = sid * 2 + cid
    g0 = wid * CHUNK
    pltpu.sync_copy(pos2_hbm.at[wid], pidx_v)
    for c in range(NSUB):
        pltpu.async_copy(moe_hbm.at[pidx_v.at[c]], row_v, sem).wait()
        pltpu.sync_copy(row_v, out_hbm.at[pl.ds(g0 + c * SUB, SUB)])


@functools.cache
def _gather():
    return pl.kernel(
        _gather_body,
        out_type=jax.ShapeDtypeStruct((N, H), jnp.float32),
        mesh=plsc.VectorSubcoreMesh(core_axis_name="c", subcore_axis_name="s"),
        scratch_types=[pltpu.VMEM((NSUB, SUB), jnp.int32),
                       pltpu.VMEM((SUB, H), jnp.float32),
                       pltpu.SemaphoreType.DMA],
        compiler_params=pltpu.CompilerParams(needs_layout_passes=False),
    )


def _moe_body(pe_ref, pt_ref, x_ref, g_ref, u_ref, d_ref, o_ref):
    f = pl.program_id(0)
    p = pl.program_id(1)
    t = pt_ref[p]
    x = x_ref[pl.ds(pl.multiple_of(t * T, T), T), :]
    g = g_ref[0]
    u = u_ref[0]
    dn = d_ref[0]
    dims = (((1,), (1,)), ((), ()))
    gg = lax.dot_general(x, g, dims, preferred_element_type=jnp.float32)
    uu = lax.dot_general(x, u, dims, preferred_element_type=jnp.float32)
    h = gg * jax.nn.sigmoid(gg) * uu
    y = lax.dot_general(h, dn, dims, preferred_element_type=jnp.float32)
    rows = pl.ds(pl.multiple_of(p * T, T), T)

    @pl.when(f == 0)
    def _():
        o_ref[rows, :] = y

    @pl.when(f > 0)
    def _():
        o_ref[rows, :] += y


def _moe_call(pair_e, pair_t, xperm, gate_w, up_w, down_w):
    grid_spec = pltpu.PrefetchScalarGridSpec(
        num_scalar_prefetch=2,
        grid=(NFB, MAXP),
        in_specs=[
            pl.BlockSpec((N, H), lambda f, p, pe, pt: (0, 0)),
            pl.BlockSpec((1, FB, H), lambda f, p, pe, pt: (pe[p], f, 0)),
            pl.BlockSpec((1, FB, H), lambda f, p, pe, pt: (pe[p], f, 0)),
            pl.BlockSpec((1, H, FB), lambda f, p, pe, pt: (pe[p], 0, f)),
        ],
        out_specs=pl.BlockSpec((MAXP * T, H), lambda f, p, pe, pt: (0, 0)),
    )
    return pl.pallas_call(
        _moe_body,
        grid_spec=grid_spec,
        out_shape=jax.ShapeDtypeStruct((MAXP * T, H), jnp.float32),
        compiler_params=pltpu.CompilerParams(
            dimension_semantics=("arbitrary", "arbitrary")),
    )(pair_e, pair_t, xperm, gate_w, up_w, down_w)


def kernel(hidden_states, experts_indices, start_indices, end_indices,
           gate_w, up_w, down_w):
    Bs, Ss, Hd = hidden_states.shape
    x = hidden_states.reshape(-1, Hd)
    idx = experts_indices.reshape(-1).astype(jnp.int32)

    oh = (idx.reshape(NW, CHUNK, 1) ==
          jnp.arange(E, dtype=jnp.int32)).astype(jnp.int32)
    ccnt = oh.sum(axis=1)
    excl = jnp.cumsum(ccnt, axis=0) - ccnt
    base = start_indices[None, :].astype(jnp.int32) + excl
    base16 = jnp.pad(base, ((0, 0), (0, L - E)))

    xperm, posn = _permute()(x, idx, base16)

    s = start_indices.astype(jnp.int32)
    en = end_indices.astype(jnp.int32)
    tt = jnp.arange(NT, dtype=jnp.int32)
    overlap = ((s[:, None] < (tt[None, :] + 1) * T)
               & (en[:, None] > tt[None, :] * T)
               & (en[:, None] > s[:, None]))
    flat = overlap.reshape(-1)
    csum = jnp.cumsum(flat.astype(jnp.int32))
    dest = jnp.where(flat, csum - 1, MAXP)
    k = jnp.arange(E * NT, dtype=jnp.int32)
    pair_e = jnp.zeros((MAXP,), jnp.int32).at[dest].set(k // NT, mode="drop")
    pair_t = jnp.zeros((MAXP,), jnp.int32).at[dest].set(k % NT, mode="drop")

    moe = _moe_call(pair_e, pair_t, xperm, gate_w, up_w, down_w)

    pair_of = (jnp.full((NT, E), 2 * MAXP, jnp.int32)
               .at[pair_t, pair_e].min(jnp.arange(MAXP, dtype=jnp.int32)))
    pos = posn.reshape(-1)
    pos2 = pair_of[pos // T, idx] * T + pos % T
    pos2 = pos2.reshape(NW, NSUB, SUB)

    y = _gather()(moe, pos2)
    return y.reshape(Bs, Ss, Hd)
